# baseline (speedup 1.0000x reference)
import functools

import jax
import jax.numpy as jnp
from jax import lax
from jax.experimental import pallas as pl
from jax.experimental.pallas import tpu as pltpu
from jax.experimental.pallas import tpu_sc as plsc

N = 10000
D = 128
NP = 10240
RW2 = 16
NWORK = 32
CH = 128
K = 79
EPAD = NWORK * K * CH
BN = 1024
RPS = NP // 16


DROW = 10016


def _hist_add(hist, dst_v, c, ones16):
    for j in range(CH // 16):
        dvec = dst_v[c, pl.ds(j * 16, 16)]
        rvec = lax.shift_right_logical(dvec, 7)
        cvec = lax.bitwise_and(dvec, 127)
        plsc.addupdate_scatter(hist, [rvec, cvec], ones16)


def _seg_body(rw, with_deg, table, *rest):
    if with_deg:
        (edges, out, e_v, dst_v, rows_v0, zbuf, hist, midx, acc, sem0) = rest
    else:
        (edges, out, e_v, dst_v, rows_v0, zbuf, acc, sem0) = rest
        hist = midx = None
    cid = lax.axis_index("c")
    sid = lax.axis_index("s")
    wid = cid * 16 + sid

    pltpu.sync_copy(edges.at[wid], e_v)

    def _unpack(i, _):
        k = i // (CH // 16)
        j = i % (CH // 16)
        p = e_v[k, pl.ds(j * 16, 16)]
        dst_v[k, pl.ds(j * 16, 16)] = lax.bitwise_and(p, 16383)
        e_v[k, pl.ds(j * 16, 16)] = lax.shift_right_logical(p, 14)
        return 0

    lax.fori_loop(0, K * (CH // 16), _unpack, 0)

    zero = jnp.zeros((16,), jnp.float32)
    ones16 = jnp.ones((16,), jnp.float32)
    for r in range(16):
        for c in range(rw // 16):
            zbuf[r, pl.ds(c * 16, 16)] = zero
    base = sid * RPS

    def _zero_step(k, _):
        pltpu.sync_copy(zbuf, acc.at[pl.ds(base + k * 16, 16)])
        return 0

    lax.fori_loop(0, RPS // 16, _zero_step, 0)

    if with_deg:
        def _zero_hist(i, _):
            r = i // (CH // 16)
            j = i % (CH // 16)
            hist[r, pl.ds(j * 16, 16)] = zero
            return 0

        lax.fori_loop(0, (NP // CH) * (CH // 16), _zero_hist, 0)
    plsc.subcore_barrier()

    def _chunk(c, _):
        pltpu.async_copy(table.at[e_v.at[c]], rows_v0, sem0).wait()
        pltpu.sync_copy(rows_v0, acc.at[dst_v.at[c]], add=True)
        if with_deg:
            _hist_add(hist, dst_v, c, ones16)
        return 0

    lax.fori_loop(0, K, _chunk, 0)
    plsc.subcore_barrier()

    if with_deg:
        def _merge(c, _):
            midx[pl.ds(0, 16)] = DROW + c * 16 + lax.iota(jnp.int32, 16)
            pltpu.sync_copy(hist.at[pl.ds(c * 16, 16)], acc.at[midx],
                            add=True)
            return 0

        lax.fori_loop(0, NP // CH // 16, _merge, 0)
        plsc.subcore_barrier()

    pltpu.sync_copy(acc.at[pl.ds(base, RPS)], out.at[cid, pl.ds(base, RPS)])


def _sc_segsum(table, edges, rw, with_deg):
    mesh = plsc.VectorSubcoreMesh(core_axis_name="c", subcore_axis_name="s")
    out_type = jax.ShapeDtypeStruct((2, NP, rw), jnp.float32)
    scratch = [
        pltpu.VMEM((K, CH), jnp.int32),
        pltpu.VMEM((K, CH), jnp.int32),
        pltpu.VMEM((CH, rw), jnp.float32),
        pltpu.VMEM((16, rw), jnp.float32),
    ]
    if with_deg:
        scratch += [
            pltpu.VMEM((NP // CH, CH), jnp.float32),
            pltpu.VMEM((16,), jnp.int32),
        ]
    scratch.append(pltpu.VMEM_SHARED((NP, rw), jnp.float32))
    scratch.append(pltpu.SemaphoreType.DMA)
    f = pl.kernel(
        functools.partial(_seg_body, rw, with_deg),
        out_type=out_type,
        mesh=mesh,
        compiler_params=pltpu.CompilerParams(use_tc_tiling_on_sc=False,
                                             needs_layout_passes=False),
        scratch_types=scratch,
    )
    return f(table, edges)


def _dense_body(x_ref, p_ref, pd_ref, w1s_ref, w1n_ref, b1_ref, w2s_ref,
                w2n_ref, b2_ref, q_ref):
    x = x_ref[...]
    a = p_ref[0] + p_ref[1]
    pd = pd_ref[...]
    deg = pd[:, 0:1] + pd[:, 1:2]
    neigh = a / jnp.maximum(deg, 1.0)
    h = jnp.dot(x, w1s_ref[...], preferred_element_type=jnp.float32)
    h += jnp.dot(neigh, w1n_ref[...], preferred_element_type=jnp.float32)
    h = jnp.maximum(h + b1_ref[...], 0.0)
    p2 = jnp.dot(h, w2n_ref[...], preferred_element_type=jnp.float32)
    hs = jnp.dot(h, w2s_ref[...], preferred_element_type=jnp.float32) + b2_ref[...]
    q_ref[...] = jnp.concatenate(
        [p2, hs, deg, jnp.zeros((x.shape[0], RW2 - 5), jnp.float32)], axis=1)


def _dense(x_pad, part1, pdeg_t, W1_self, W1_neigh, b1, W2_self, W2_neigh, b2):
    grid = (NP // BN,)
    return pl.pallas_call(
        _dense_body,
        grid=grid,
        in_specs=[
            pl.BlockSpec((BN, D), lambda i: (i, 0)),
            pl.BlockSpec((2, BN, D), lambda i: (0, i, 0)),
            pl.BlockSpec((BN, 2), lambda i: (i, 0)),
            pl.BlockSpec((D, D), lambda i: (0, 0)),
            pl.BlockSpec((D, D), lambda i: (0, 0)),
            pl.BlockSpec((1, D), lambda i: (0, 0)),
            pl.BlockSpec((D, 2), lambda i: (0, 0)),
            pl.BlockSpec((D, 2), lambda i: (0, 0)),
            pl.BlockSpec((1, 2), lambda i: (0, 0)),
        ],
        out_specs=pl.BlockSpec((BN, RW2), lambda i: (i, 0)),
        out_shape=jax.ShapeDtypeStruct((NP, RW2), jnp.float32),
    )(x_pad, part1, pdeg_t, W1_self, W1_neigh, b1.reshape(1, D), W2_self,
      W2_neigh, b2.reshape(1, 2))


def _combine_body(q_ref, p2_ref, out_ref):
    q = q_ref[...]
    a = p2_ref[0] + p2_ref[1]
    deg = jnp.maximum(q[:, 4:5], 1.0)
    out_ref[...] = q[:, 2:4] + a[:, 0:2] / deg


def _combine(q, part2):
    grid = (NP // BN,)
    return pl.pallas_call(
        _combine_body,
        grid=grid,
        in_specs=[
            pl.BlockSpec((BN, RW2), lambda i: (i, 0)),
            pl.BlockSpec((2, BN, RW2), lambda i: (0, i, 0)),
        ],
        out_specs=pl.BlockSpec((BN, 2), lambda i: (i, 0)),
        out_shape=jax.ShapeDtypeStruct((NP, 2), jnp.float32),
    )(q, part2)


def kernel(x, edge_index, W1_self, W1_neigh, b1, W2_self, W2_neigh, b2):
    src = edge_index[0].astype(jnp.int32)
    dst = edge_index[1].astype(jnp.int32)
    E = src.shape[0]

    packed = jnp.left_shift(src, 14) | dst
    edges = jnp.full((EPAD,), N, jnp.int32).at[:E].set(packed)
    edges = edges.reshape(NWORK, K, CH)

    x_pad = jnp.zeros((NP, D), jnp.float32).at[:N].set(x)

    part1 = _sc_segsum(x_pad, edges, D, True)
    pdeg = part1[:, DROW:DROW + NP // CH, :].reshape(2, NP)
    q = _dense(x_pad, part1, pdeg.T, W1_self, W1_neigh, b1, W2_self, W2_neigh,
               b2)
    part2 = _sc_segsum(q, edges, RW2, False)
    out = _combine(q, part2)
    return out[:N]
